# baseline (speedup 1.0000x reference)
import functools

import jax
import jax.numpy as jnp
from jax import lax
from jax.experimental import pallas as pl
from jax.experimental.pallas import tpu as pltpu
from jax.experimental.pallas import tpu_sc as plsc

_VOCAB = 1000000
_PAD = _PAD_IDX = _VOCAB - 1
_L = 16


@functools.cache
def _sc_info():
    info = plsc.get_sparse_core_info()
    return info.num_cores, info.num_subcores


@functools.cache
def _make_relayout(vocab, emb, rows_per_buf):
    nc, ns = _sc_info()
    nw = nc * ns
    n128 = vocab * emb // 128
    nbuf = n128 // rows_per_buf
    r4 = 128 // emb
    rem = nbuf % nw
    mesh = plsc.VectorSubcoreMesh(core_axis_name="c", subcore_axis_name="s")

    @functools.partial(
        pl.kernel,
        mesh=mesh,
        out_type=jax.ShapeDtypeStruct((n128, 128), jnp.float32),
        scratch_types=[
            pltpu.VMEM((rows_per_buf * r4, emb), jnp.float32),
            pltpu.VMEM((rows_per_buf, 128), jnp.float32),
        ],
    )
    def k(table_hbm, out_hbm, in_v, out_v):
        wid = lax.axis_index("s") * nc + lax.axis_index("c")
        cnt = (nbuf // nw) + jnp.where(wid < rem, 1, 0)

        def body(ci, carry):
            b = wid + ci * nw
            r = pl.multiple_of(b * rows_per_buf, 8)
            pltpu.sync_copy(table_hbm.at[pl.ds(r * r4, rows_per_buf * r4)],
                            in_v)

            def regroup(g, c2):
                for p in range(r4):
                    for h in range(emb // _L):
                        out_v[g, pl.ds(p * emb + h * _L, _L)] = (
                            in_v[g * r4 + p, pl.ds(h * _L, _L)])
                return c2

            lax.fori_loop(0, rows_per_buf, regroup, 0)
            pltpu.sync_copy(out_v, out_hbm.at[pl.ds(r, rows_per_buf)])
            return carry

        lax.fori_loop(0, cnt, body, 0)

    return k


@functools.cache
def _make_gather(n_seq, seq_len, vocab, emb, seq_per_chunk):
    nc, ns = _sc_info()
    nw = nc * ns
    seq_per_w = n_seq // nw
    nchunk = seq_per_w // seq_per_chunk
    chunk = seq_per_chunk * seq_len
    blk = 128 // emb
    mesh = plsc.VectorSubcoreMesh(core_axis_name="c", subcore_axis_name="s")

    @functools.partial(
        pl.kernel,
        mesh=mesh,
        out_type=jax.ShapeDtypeStruct((n_seq, seq_len, emb), jnp.float32),
        scratch_types=[
            pltpu.VMEM((chunk,), jnp.int32),
            pltpu.VMEM((chunk,), jnp.int32),
            pltpu.VMEM((chunk, 128), jnp.float32),
            pltpu.VMEM((chunk, emb), jnp.float32),
            pltpu.SemaphoreType.DMA,
        ],
    )
    def k(tokens_hbm, table_hbm, out_hbm, idx_v, idx4_v, blk_v, out_v, sem):
        wid = lax.axis_index("s") * nc + lax.axis_index("c")
        wseq = wid * seq_per_w

        def chunk_body(ci, carry):
            sq0 = wseq + ci * seq_per_chunk
            base = pl.multiple_of(sq0 * seq_len, 8)
            pltpu.sync_copy(tokens_hbm.at[pl.ds(base, chunk)], idx_v)

            def prep_body(g, acc):
                v = idx_v[pl.ds(g * _L, _L)]
                idx4_v[pl.ds(g * _L, _L)] = v >> 2
                return acc | jnp.where(v == _PAD, 1, 0)

            acc = lax.fori_loop(0, chunk // _L, prep_body,
                                jnp.zeros((_L,), jnp.int32))
            cp = pltpu.async_copy(table_hbm.at[idx4_v], blk_v, sem)

            for sh in (8, 4, 2, 1):
                perm = lax.iota(jnp.int32, _L) ^ sh
                acc = acc | acc.at[perm].get(mode="promise_in_bounds")
            npad = acc[0]
            cp.wait()

            def compact_body(g, c2):
                v = idx_v[pl.ds(g * _L, _L)]
                for j in range(_L):
                    off = (v[j] & (blk - 1)) * emb
                    r = g * _L + j
                    for h in range(emb // _L):
                        out_v[r, pl.ds(h * _L, _L)] = (
                            blk_v[r, pl.ds(off + h * _L, _L)])
                return c2

            lax.fori_loop(0, chunk // _L, compact_body, 0)

            @pl.when(npad > 0)
            def _fix_pads():
                zeros = jnp.zeros((_L,), jnp.float32)

                def fix_group(g, c2):
                    v = idx_v[pl.ds(g * _L, _L)]
                    gacc = jnp.where(v == _PAD, 1, 0)
                    for sh in (8, 4, 2, 1):
                        perm = lax.iota(jnp.int32, _L) ^ sh
                        gacc = gacc | gacc.at[perm].get(
                            mode="promise_in_bounds")

                    @pl.when(gacc[0] > 0)
                    def _():
                        for j in range(_L):
                            @pl.when(v[j] == _PAD)
                            def _zero_row(j=j):
                                r = g * _L + j
                                for h in range(emb // _L):
                                    out_v[r, pl.ds(h * _L, _L)] = zeros

                    return c2

                lax.fori_loop(0, chunk // _L, fix_group, 0)

            for q in range(seq_per_chunk):
                pltpu.sync_copy(out_v.at[pl.ds(q * seq_len, seq_len)],
                                out_hbm.at[sq0 + q])
            return carry

        lax.fori_loop(0, nchunk, chunk_body, 0)

    return k


@jax.jit
def kernel(tokens, table):
    n_seq, seq_len = tokens.shape
    vocab, emb = table.shape
    table128 = _make_relayout(vocab, emb, 200)(table)
    return _make_gather(n_seq, seq_len, vocab, emb, 2)(
        tokens.reshape(-1), table128)


# speedup vs baseline: 1.6586x; 1.6586x over previous
import functools

import jax
import jax.numpy as jnp
from jax import lax
from jax.experimental import pallas as pl
from jax.experimental.pallas import tpu as pltpu
from jax.experimental.pallas import tpu_sc as plsc

_VOCAB = 1000000
_PAD = _VOCAB - 1
_L = 16


@functools.cache
def _make_sc_embed(n_rows, vocab, emb, chunk):
    info = plsc.get_sparse_core_info()
    nc, ns = info.num_cores, info.num_subcores
    nw = nc * ns
    rows_per_w = n_rows // nw
    nchunk = rows_per_w // chunk
    mesh = plsc.VectorSubcoreMesh(core_axis_name="c", subcore_axis_name="s")

    @functools.partial(
        pl.kernel,
        mesh=mesh,
        out_type=jax.ShapeDtypeStruct((n_rows, 128), jnp.float32),
        scratch_types=[
            pltpu.VMEM((chunk,), jnp.int32),
            pltpu.VMEM((chunk, 128), jnp.float32),
            pltpu.SemaphoreType.DMA,
        ],
    )
    def k(tokens_hbm, table_hbm, out_hbm, idx_v, rows_v, sem):
        wid = lax.axis_index("s") * nc + lax.axis_index("c")
        wbase = wid * rows_per_w

        def chunk_body(ci, carry):
            base = pl.multiple_of(wbase + ci * chunk, 8)
            pltpu.sync_copy(tokens_hbm.at[pl.ds(base, chunk)], idx_v)
            cp = pltpu.async_copy(table_hbm.at[idx_v], rows_v, sem)

            def scan_body(g, acc):
                v = idx_v[pl.ds(g * _L, _L)]
                return acc | jnp.where(v == _PAD, 1, 0)

            acc = lax.fori_loop(0, chunk // _L, scan_body,
                                jnp.zeros((_L,), jnp.int32))
            for sh in (8, 4, 2, 1):
                perm = lax.iota(jnp.int32, _L) ^ sh
                acc = acc | acc.at[perm].get(mode="promise_in_bounds")
            npad = acc[0]
            cp.wait()

            @pl.when(npad > 0)
            def _fix_pads():
                zeros = jnp.zeros((_L,), jnp.float32)

                def fix_group(g, c2):
                    v = idx_v[pl.ds(g * _L, _L)]
                    gacc = jnp.where(v == _PAD, 1, 0)
                    for sh in (8, 4, 2, 1):
                        perm = lax.iota(jnp.int32, _L) ^ sh
                        gacc = gacc | gacc.at[perm].get(
                            mode="promise_in_bounds")

                    @pl.when(gacc[0] > 0)
                    def _():
                        for j in range(_L):
                            @pl.when(v[j] == _PAD)
                            def _zero_row(j=j):
                                r = g * _L + j
                                for h in range(emb // _L):
                                    rows_v[r, pl.ds(h * _L, _L)] = zeros

                    return c2

                lax.fori_loop(0, chunk // _L, fix_group, 0)

            pltpu.sync_copy(rows_v, out_hbm.at[pl.ds(base, chunk)])
            return carry

        lax.fori_loop(0, nchunk, chunk_body, 0)

    return k


@jax.jit
def kernel(tokens, table):
    n_seq, seq_len = tokens.shape
    vocab, emb = table.shape
    table_p = jnp.pad(table, ((0, 0), (0, 128 - emb)))
    flat = tokens.reshape(-1)
    out = _make_sc_embed(flat.shape[0], vocab, emb, 800)(flat, table_p)
    return out[:, :emb].reshape(n_seq, seq_len, emb)


# speedup vs baseline: 1.6729x; 1.0086x over previous
import functools

import jax
import jax.numpy as jnp
from jax import lax
from jax.experimental import pallas as pl
from jax.experimental.pallas import tpu as pltpu
from jax.experimental.pallas import tpu_sc as plsc

_VOCAB = 1000000
_PAD = _VOCAB - 1
_L = 16


@functools.cache
def _make_sc_embed(n_rows, vocab, emb, chunk):
    info = plsc.get_sparse_core_info()
    nc, ns = info.num_cores, info.num_subcores
    nw = nc * ns
    rows_per_w = n_rows // nw
    nchunk = rows_per_w // chunk
    mesh = plsc.VectorSubcoreMesh(core_axis_name="c", subcore_axis_name="s")

    @functools.partial(
        pl.kernel,
        mesh=mesh,
        out_type=jax.ShapeDtypeStruct((n_rows, 128), jnp.float32),
        scratch_types=[
            pltpu.VMEM((chunk,), jnp.int32),
            pltpu.VMEM((chunk,), jnp.int32),
            pltpu.VMEM((chunk, 128), jnp.float32),
            pltpu.VMEM((chunk, 128), jnp.float32),
            pltpu.SemaphoreType.DMA,
            pltpu.SemaphoreType.DMA,
        ],
    )
    def k(tokens_hbm, table_hbm, out_hbm, idx0_v, idx1_v, rows0_v, rows1_v,
          gsem, wsem):
        wid = lax.axis_index("s") * nc + lax.axis_index("c")
        wbase = wid * rows_per_w
        bufs = ((idx0_v, rows0_v), (idx1_v, rows1_v))

        def process(ci, idx_v, rows_v, first):
            base = pl.multiple_of(wbase + ci * chunk, 8)
            pltpu.sync_copy(tokens_hbm.at[pl.ds(base, chunk)], idx_v)
            cp = pltpu.async_copy(table_hbm.at[idx_v], rows_v, gsem)

            def scan_body(g, acc):
                v = idx_v[pl.ds(g * _L, _L)]
                return acc | jnp.where(v == _PAD, 1, 0)

            acc = lax.fori_loop(0, chunk // _L, scan_body,
                                jnp.zeros((_L,), jnp.int32))
            for sh in (8, 4, 2, 1):
                perm = lax.iota(jnp.int32, _L) ^ sh
                acc = acc | acc.at[perm].get(mode="promise_in_bounds")
            npad = acc[0]
            cp.wait()

            @pl.when(npad > 0)
            def _fix_pads():
                zeros = jnp.zeros((_L,), jnp.float32)

                def fix_group(g, c2):
                    v = idx_v[pl.ds(g * _L, _L)]
                    gacc = jnp.where(v == _PAD, 1, 0)
                    for sh in (8, 4, 2, 1):
                        perm = lax.iota(jnp.int32, _L) ^ sh
                        gacc = gacc | gacc.at[perm].get(
                            mode="promise_in_bounds")

                    @pl.when(gacc[0] > 0)
                    def _():
                        for j in range(_L):
                            @pl.when(v[j] == _PAD)
                            def _zero_row(j=j):
                                r = g * _L + j
                                for h in range(emb // _L):
                                    rows_v[r, pl.ds(h * _L, _L)] = zeros

                    return c2

                lax.fori_loop(0, chunk // _L, fix_group, 0)

            @pl.when(jnp.logical_not(first))
            def _drain():
                pltpu.make_async_copy(
                    rows_v, out_hbm.at[pl.ds(base, chunk)], wsem).wait()

            pltpu.async_copy(rows_v, out_hbm.at[pl.ds(base, chunk)], wsem)

        def pair_body(pi, carry):
            for b in range(2):
                (idx_v, rows_v) = bufs[b]
                process(pi * 2 + b, idx_v, rows_v, pi == 0)
            return carry

        lax.fori_loop(0, nchunk // 2, pair_body, 0)
        last = pl.multiple_of(wbase, 8)
        for _ in range(2):
            pltpu.make_async_copy(
                rows0_v, out_hbm.at[pl.ds(last, chunk)], wsem).wait()

    return k


@jax.jit
def kernel(tokens, table):
    n_seq, seq_len = tokens.shape
    vocab, emb = table.shape
    table_p = jnp.pad(table, ((0, 0), (0, 128 - emb)))
    flat = tokens.reshape(-1)
    out = _make_sc_embed(flat.shape[0], vocab, emb, 400)(flat, table_p)
    return out[:, :emb].reshape(n_seq, seq_len, emb)
